# baseline (speedup 1.0000x reference)
import numpy as np
import jax
import jax.numpy as jnp
from jax.experimental import pallas as pl

_P = 1.0
_FREQ_MASK_PARAM = 27
_TIME_MASK_PARAM = 100
_FREQ_MASKS = 2
_TIME_MASKS = 2


def _mask_vectors(batch, n_freq, n_time):
    rng = np.random.default_rng(0)
    if rng.random() > _P:
        return None
    rowm = np.zeros((batch, n_freq), np.float32)
    colm = np.zeros((batch, n_time), np.float32)
    for idx in range(batch):
        for _ in range(_FREQ_MASKS):
            max_w = min(_FREQ_MASK_PARAM, n_freq)
            w = int(rng.integers(0, max_w + 1))
            if w > 0:
                s = int(rng.integers(0, n_freq - w + 1))
                rowm[idx, s:s + w] = 1.0
        for _ in range(_TIME_MASKS):
            max_w = min(_TIME_MASK_PARAM, n_time)
            w = int(rng.integers(0, max_w + 1))
            if w > 0:
                s = int(rng.integers(0, n_time - w + 1))
                colm[idx, s:s + w] = 1.0
    return rowm, colm


def _body(x_ref, rowm_ref, colm_ref, o_ref):
    xb = x_ref[0, 0]
    fill = jnp.mean(xb)
    rm = rowm_ref[0, 0, :]
    cm = colm_ref[0, 0, :]
    m = jnp.maximum(rm[:, None], cm[None, :]) > 0.0
    o_ref[0, 0] = jnp.where(m, fill, xb)


def kernel(x):
    batch, ch, n_freq, n_time = x.shape
    masks = _mask_vectors(batch, n_freq, n_time)
    if masks is None:
        return x
    rowm_np, colm_np = masks
    rowm = jnp.asarray(rowm_np).reshape(batch, 1, n_freq)
    colm = jnp.asarray(colm_np).reshape(batch, 1, n_time)

    out = pl.pallas_call(
        _body,
        grid=(batch,),
        in_specs=[
            pl.BlockSpec((1, ch, n_freq, n_time), lambda b: (b, 0, 0, 0)),
            pl.BlockSpec((1, 1, n_freq), lambda b: (b, 0, 0)),
            pl.BlockSpec((1, 1, n_time), lambda b: (b, 0, 0)),
        ],
        out_specs=pl.BlockSpec((1, ch, n_freq, n_time), lambda b: (b, 0, 0, 0)),
        out_shape=jax.ShapeDtypeStruct(x.shape, x.dtype),
    )(x, rowm, colm)
    return out


# speedup vs baseline: 1.1144x; 1.1144x over previous
import numpy as np
import jax
import jax.numpy as jnp
from jax.experimental import pallas as pl
from jax.experimental.pallas import tpu as pltpu

_P = 1.0
_FREQ_MASK_PARAM = 27
_TIME_MASK_PARAM = 100
_FREQ_MASKS = 2
_TIME_MASKS = 2

_NBUF = 6
_LEAD = 3
_NQ = 8


def _mask_vectors(batch, n_freq, n_time):
    rng = np.random.default_rng(0)
    if rng.random() > _P:
        return None
    rowm = np.zeros((batch, n_freq), np.float32)
    colm = np.zeros((batch, n_time), np.float32)
    for idx in range(batch):
        for _ in range(_FREQ_MASKS):
            max_w = min(_FREQ_MASK_PARAM, n_freq)
            w = int(rng.integers(0, max_w + 1))
            if w > 0:
                s = int(rng.integers(0, n_freq - w + 1))
                rowm[idx, s:s + w] = 1.0
        for _ in range(_TIME_MASKS):
            max_w = min(_TIME_MASK_PARAM, n_time)
            w = int(rng.integers(0, max_w + 1))
            if w > 0:
                s = int(rng.integers(0, n_time - w + 1))
                colm[idx, s:s + w] = 1.0
    return rowm, colm


def _make_body(batch, n_freq, n_time):
    n_elem = float(n_freq * n_time)
    rows_per = n_freq // _NQ

    def _load_chunks(x_hbm, buf, sems, j, k, do_start):
        for q in range(_NQ):
            cp = pltpu.make_async_copy(
                x_hbm.at[j, 0, pl.ds(q * rows_per, rows_per)],
                buf.at[k, pl.ds(q * rows_per, rows_per)],
                sems.at[k, q],
            )
            if do_start:
                cp.start()
            else:
                cp.wait()

    def _store_chunks(o_hbm, buf, sems, j, k, do_start):
        for q in range(_NQ):
            cp = pltpu.make_async_copy(
                buf.at[k, pl.ds(q * rows_per, rows_per)],
                o_hbm.at[j, 0, pl.ds(q * rows_per, rows_per)],
                sems.at[k, q],
            )
            if do_start:
                cp.start()
            else:
                cp.wait()

    def _body(x_hbm, rowm_ref, colm_ref, o_hbm, buf, load_sems, store_sems):
        i = pl.program_id(0)

        @pl.when(i < batch)
        def _issue_load():
            k = jax.lax.rem(i, _NBUF)

            @pl.when(i >= _NBUF)
            def _():
                _store_chunks(o_hbm, buf, store_sems, i - _NBUF, k, False)

            _load_chunks(x_hbm, buf, load_sems, i, k, True)

        j = i - _LEAD

        @pl.when(j >= 0)
        def _process():
            k = jax.lax.rem(j, _NBUF)
            _load_chunks(x_hbm, buf, load_sems, j, k, False)
            xb = buf[k]
            fill = jnp.sum(xb) * (1.0 / n_elem)
            rm = rowm_ref[j, 0, :]
            cm = colm_ref[j, 0, :]
            m = jnp.maximum(rm[:, None], cm[None, :]) > 0.0
            buf[k] = jnp.where(m, fill, xb)
            _store_chunks(o_hbm, buf, store_sems, j, k, True)

        @pl.when(i == batch + _LEAD - 1)
        def _drain():
            for d in range(_NBUF):
                jj = batch - _NBUF + d
                _store_chunks(o_hbm, buf, store_sems, jj, jj % _NBUF, False)

    return _body


def kernel(x):
    batch, ch, n_freq, n_time = x.shape
    masks = _mask_vectors(batch, n_freq, n_time)
    if masks is None:
        return x
    rowm_np, colm_np = masks
    rowm = jnp.asarray(rowm_np).reshape(batch, 1, n_freq)
    colm = jnp.asarray(colm_np).reshape(batch, 1, n_time)

    out = pl.pallas_call(
        _make_body(batch, n_freq, n_time),
        grid=(batch + _LEAD,),
        in_specs=[
            pl.BlockSpec(memory_space=pltpu.MemorySpace.HBM),
            pl.BlockSpec(memory_space=pltpu.MemorySpace.VMEM),
            pl.BlockSpec(memory_space=pltpu.MemorySpace.VMEM),
        ],
        out_specs=pl.BlockSpec(memory_space=pltpu.MemorySpace.HBM),
        out_shape=jax.ShapeDtypeStruct(x.shape, x.dtype),
        scratch_shapes=[
            pltpu.VMEM((_NBUF, n_freq, n_time), x.dtype),
            pltpu.SemaphoreType.DMA((_NBUF, _NQ)),
            pltpu.SemaphoreType.DMA((_NBUF, _NQ)),
        ],
    )(x, rowm, colm)
    return out
